# Initial kernel scaffold; baseline (speedup 1.0000x reference)
import jax
import jax.numpy as jnp
from jax.experimental import pallas as pl


def kernel(hidden_states, router_w, gate_w, up_w, down_w):
    raise NotImplementedError("write your pallas kernel here")


# speedup vs baseline: 1.4169x; 1.4169x over previous
import functools
import jax
import jax.numpy as jnp
from jax import lax
from jax.experimental import pallas as pl
from jax.experimental.pallas import tpu as pltpu

S, D, F, E = 2048, 1024, 4096, 8
A = S * 2
BLK = 256
NB = A // BLK + E - 1
NBP = 32
NSLOT = NB * BLK
FB = 512
NF = F // FB
LANES = 128
NEG = -1e30


def _routing_kernel(x_ref, rw_ref, logits_ref, misc_ref, be_ref,
                    oh1_s, oh2_s, r1_s, r2_s):
    x = x_ref[...]
    logits = jnp.dot(x, rw_ref[...], preferred_element_type=jnp.float32)
    logits_ref[...] = logits
    col = lax.broadcasted_iota(jnp.int32, (S, LANES), 1)
    neg = jnp.where(col < E, logits, NEG)
    m1 = jnp.max(neg, axis=1, keepdims=True)
    i1 = jnp.min(jnp.where(neg == m1, col, LANES), axis=1, keepdims=True)
    masked = jnp.where(col == i1, NEG, neg)
    m2 = jnp.max(masked, axis=1, keepdims=True)
    i2 = jnp.min(jnp.where(masked == m2, col, LANES), axis=1, keepdims=True)
    w1 = 1.0 / (1.0 + jnp.exp(m2 - m1))
    w2 = 1.0 - w1
    oh1 = (col == i1).astype(jnp.float32)
    oh2 = (col == i2).astype(jnp.float32)
    oh1_s[...] = oh1
    oh2_s[...] = oh2
    row128 = lax.broadcasted_iota(jnp.int32, (128, 128), 0)
    col128 = lax.broadcasted_iota(jnp.int32, (128, 128), 1)
    tlow = (row128 > col128).astype(jnp.float32)

    def mk_body(oh_s, r_s):
        def body(c, carry):
            ch = oh_s[pl.ds(c * 128, 128), :]
            rk = carry + jnp.dot(tlow, ch, preferred_element_type=jnp.float32)
            r_s[pl.ds(c * 128, 128), :] = jnp.sum(rk * ch, axis=1, keepdims=True)
            return carry + jnp.sum(ch, axis=0, keepdims=True)
        return body

    carry = lax.fori_loop(0, S // 128, mk_body(oh1_s, r1_s),
                          jnp.zeros((1, LANES), jnp.float32))
    counts = lax.fori_loop(0, S // 128, mk_body(oh2_s, r2_s), carry)
    blocks = jnp.floor((counts + (BLK - 1.0)) / BLK)
    ustrict = (row128 < col128).astype(jnp.float32)
    excl = jnp.dot(blocks, ustrict, preferred_element_type=jnp.float32)
    incl = excl + blocks
    pad_off = excl * BLK
    used = jnp.sum(blocks)
    slot1 = r1_s[...] + jnp.sum(oh1 * pad_off, axis=1, keepdims=True)
    slot2 = r2_s[...] + jnp.sum(oh2 * pad_off, axis=1, keepdims=True)
    misc = (jnp.where(col == 0, slot1, 0.0) + jnp.where(col == 1, slot2, 0.0)
            + jnp.where(col == 2, w1, 0.0) + jnp.where(col == 3, w2, 0.0))
    misc_ref[...] = misc
    jb = lax.broadcasted_iota(jnp.int32, (NBP, LANES), 0).astype(jnp.float32)
    jj = jnp.minimum(jb, used - 1.0)
    be_row = jnp.sum((jj >= incl).astype(jnp.float32), axis=1, keepdims=True)
    colb = lax.broadcasted_iota(jnp.int32, (NBP, LANES), 1)
    be_ref[...] = (jnp.where(colb == 0, be_row, 0.0)
                   + jnp.where(colb == 1, used, 0.0))


def _routing(x, rw_pad):
    return pl.pallas_call(
        _routing_kernel,
        out_shape=(
            jax.ShapeDtypeStruct((S, LANES), jnp.float32),
            jax.ShapeDtypeStruct((S, LANES), jnp.float32),
            jax.ShapeDtypeStruct((NBP, LANES), jnp.float32),
        ),
        scratch_shapes=[
            pltpu.VMEM((S, LANES), jnp.float32),
            pltpu.VMEM((S, LANES), jnp.float32),
            pltpu.VMEM((S, 1), jnp.float32),
            pltpu.VMEM((S, 1), jnp.float32),
        ],
    )(x, rw_pad)


def _moe_kernel(be_sref, nb_sref, x_ref, w_ref, gate_ref, up_ref, down_ref,
                out_ref):
    f = pl.program_id(0)
    i = pl.program_id(1)

    @pl.when(i < nb_sref[0])
    def _():
        rows = pl.ds(i * BLK, BLK)
        x = x_ref[rows, :]
        g = jnp.dot(x, gate_ref[0], preferred_element_type=jnp.float32)
        u = jnp.dot(x, up_ref[0], preferred_element_type=jnp.float32)
        h = jax.nn.gelu(g, approximate=True) * u * w_ref[0]
        contrib = jnp.dot(h, down_ref[0], preferred_element_type=jnp.float32)

        @pl.when(f == 0)
        def _():
            out_ref[rows, :] = contrib

        @pl.when(f > 0)
        def _():
            out_ref[rows, :] = out_ref[rows, :] + contrib


def _moe(be, nb, xs, w3, gate_w, up_w, down_w):
    grid_spec = pltpu.PrefetchScalarGridSpec(
        num_scalar_prefetch=2,
        grid=(NF, NB),
        in_specs=[
            pl.BlockSpec((NSLOT, D), lambda f, i, be, nb: (0, 0)),
            pl.BlockSpec((1, BLK, 1), lambda f, i, be, nb: (i, 0, 0)),
            pl.BlockSpec((1, D, FB), lambda f, i, be, nb: (be[i], 0, f)),
            pl.BlockSpec((1, D, FB), lambda f, i, be, nb: (be[i], 0, f)),
            pl.BlockSpec((1, FB, D), lambda f, i, be, nb: (be[i], f, 0)),
        ],
        out_specs=pl.BlockSpec((NSLOT, D), lambda f, i, be, nb: (0, 0)),
    )
    return pl.pallas_call(
        _moe_kernel,
        grid_spec=grid_spec,
        out_shape=jax.ShapeDtypeStruct((NSLOT, D), jnp.float32),
        compiler_params=pltpu.CompilerParams(
            dimension_semantics=("arbitrary", "arbitrary"),
            vmem_limit_bytes=120 * 1024 * 1024,
        ),
    )(be, nb, xs, w3, gate_w, up_w, down_w)


def kernel(hidden_states, router_w, gate_w, up_w, down_w):
    x = hidden_states.reshape(S, D)
    rw_pad = jnp.pad(router_w, ((0, 0), (0, LANES - E)))
    logits_pad, misc, be_pack = _routing(x, rw_pad)
    slot1 = misc[:, 0].astype(jnp.int32)
    slot2 = misc[:, 1].astype(jnp.int32)
    w1 = misc[:, 2]
    w2 = misc[:, 3]
    be = be_pack[:, 0].astype(jnp.int32)
    nb = be_pack[:1, 1].astype(jnp.int32)

    xs = jnp.zeros((NSLOT, D), jnp.float32)
    xs = xs.at[slot1].set(x)
    xs = xs.at[slot2].set(x)
    ws = jnp.zeros((NSLOT,), jnp.float32)
    ws = ws.at[slot1].set(w1)
    ws = ws.at[slot2].set(w2)

    w3 = ws.reshape(NB, BLK, 1)
    ys = _moe(be, nb, xs, w3, gate_w, up_w, down_w)

    out = ys[slot1] + ys[slot2]

    return (out.reshape(1, S, D), logits_pad[:, :E].reshape(1, S, E))


# speedup vs baseline: 1.6228x; 1.1453x over previous
import functools
import jax
import jax.numpy as jnp
from jax import lax
from jax.experimental import pallas as pl
from jax.experimental.pallas import tpu as pltpu
from jax.experimental.pallas import tpu_sc as plsc

S, D, F, E = 2048, 1024, 4096, 8
A = S * 2
BLK = 256
NB = A // BLK + E - 1
NBP = 32
NSLOT = NB * BLK
FB = 512
NF = F // FB
LANES = 128
NEG = -1e30


def _routing_kernel(x_ref, rw_ref, logits_ref, misc_ref, be_ref,
                    oh1_s, oh2_s, r1_s, r2_s):
    x = x_ref[...]
    logits = jnp.dot(x, rw_ref[...], preferred_element_type=jnp.float32)
    logits_ref[...] = logits
    col = lax.broadcasted_iota(jnp.int32, (S, LANES), 1)
    neg = jnp.where(col < E, logits, NEG)
    m1 = jnp.max(neg, axis=1, keepdims=True)
    i1 = jnp.min(jnp.where(neg == m1, col, LANES), axis=1, keepdims=True)
    masked = jnp.where(col == i1, NEG, neg)
    m2 = jnp.max(masked, axis=1, keepdims=True)
    i2 = jnp.min(jnp.where(masked == m2, col, LANES), axis=1, keepdims=True)
    w1 = 1.0 / (1.0 + jnp.exp(m2 - m1))
    w2 = 1.0 - w1
    oh1 = (col == i1).astype(jnp.float32)
    oh2 = (col == i2).astype(jnp.float32)
    oh1_s[...] = oh1
    oh2_s[...] = oh2
    row128 = lax.broadcasted_iota(jnp.int32, (128, 128), 0)
    col128 = lax.broadcasted_iota(jnp.int32, (128, 128), 1)
    tlow = (row128 > col128).astype(jnp.float32)

    def mk_body(oh_s, r_s):
        def body(c, carry):
            ch = oh_s[pl.ds(c * 128, 128), :]
            rk = carry + jnp.dot(tlow, ch, preferred_element_type=jnp.float32)
            r_s[pl.ds(c * 128, 128), :] = jnp.sum(rk * ch, axis=1, keepdims=True)
            return carry + jnp.sum(ch, axis=0, keepdims=True)
        return body

    carry = lax.fori_loop(0, S // 128, mk_body(oh1_s, r1_s),
                          jnp.zeros((1, LANES), jnp.float32))
    counts = lax.fori_loop(0, S // 128, mk_body(oh2_s, r2_s), carry)
    blocks = jnp.floor((counts + (BLK - 1.0)) / BLK)
    ustrict = (row128 < col128).astype(jnp.float32)
    excl = jnp.dot(blocks, ustrict, preferred_element_type=jnp.float32)
    incl = excl + blocks
    pad_off = excl * BLK
    used = jnp.sum(blocks)
    slot1 = r1_s[...] + jnp.sum(oh1 * pad_off, axis=1, keepdims=True)
    slot2 = r2_s[...] + jnp.sum(oh2 * pad_off, axis=1, keepdims=True)
    misc = (jnp.where(col == 0, slot1, 0.0) + jnp.where(col == 1, slot2, 0.0)
            + jnp.where(col == 2, w1, 0.0) + jnp.where(col == 3, w2, 0.0))
    misc_ref[...] = misc
    jb = lax.broadcasted_iota(jnp.int32, (NBP, LANES), 0).astype(jnp.float32)
    jj = jnp.minimum(jb, used - 1.0)
    be_row = jnp.sum((jj >= incl).astype(jnp.float32), axis=1, keepdims=True)
    colb = lax.broadcasted_iota(jnp.int32, (NBP, LANES), 1)
    be_ref[...] = (jnp.where(colb == 0, be_row, 0.0)
                   + jnp.where(colb == 1, used, 0.0))


def _routing(x, rw_pad):
    return pl.pallas_call(
        _routing_kernel,
        out_shape=(
            jax.ShapeDtypeStruct((S, LANES), jnp.float32),
            jax.ShapeDtypeStruct((S, LANES), jnp.float32),
            jax.ShapeDtypeStruct((NBP, LANES), jnp.float32),
        ),
        scratch_shapes=[
            pltpu.VMEM((S, LANES), jnp.float32),
            pltpu.VMEM((S, LANES), jnp.float32),
            pltpu.VMEM((S, 1), jnp.float32),
            pltpu.VMEM((S, 1), jnp.float32),
        ],
    )(x, rw_pad)


def _moe_kernel(be_sref, nb_sref, x_ref, w_ref, gate_ref, up_ref, down_ref,
                out_ref):
    f = pl.program_id(0)
    i = pl.program_id(1)

    @pl.when(i < nb_sref[0])
    def _():
        rows = pl.ds(i * BLK, BLK)
        x = x_ref[rows, :]
        g = jnp.dot(x, gate_ref[0], preferred_element_type=jnp.float32)
        u = jnp.dot(x, up_ref[0], preferred_element_type=jnp.float32)
        h = jax.nn.gelu(g, approximate=True) * u * w_ref[:, 0:1]
        contrib = jnp.dot(h, down_ref[0], preferred_element_type=jnp.float32)

        @pl.when(f == 0)
        def _():
            out_ref[rows, :] = contrib

        @pl.when(f > 0)
        def _():
            out_ref[rows, :] = out_ref[rows, :] + contrib


def _moe(be, nb, xs, w3, gate_w, up_w, down_w):
    grid_spec = pltpu.PrefetchScalarGridSpec(
        num_scalar_prefetch=2,
        grid=(NF, NB),
        in_specs=[
            pl.BlockSpec((NSLOT, D), lambda f, i, be, nb: (0, 0)),
            pl.BlockSpec((BLK, WV), lambda f, i, be, nb: (i, 0)),
            pl.BlockSpec((1, D, FB), lambda f, i, be, nb: (be[i], 0, f)),
            pl.BlockSpec((1, D, FB), lambda f, i, be, nb: (be[i], 0, f)),
            pl.BlockSpec((1, FB, D), lambda f, i, be, nb: (be[i], f, 0)),
        ],
        out_specs=pl.BlockSpec((NSLOT, D), lambda f, i, be, nb: (0, 0)),
    )
    return pl.pallas_call(
        _moe_kernel,
        grid_spec=grid_spec,
        out_shape=jax.ShapeDtypeStruct((NSLOT, D), jnp.float32),
        compiler_params=pltpu.CompilerParams(
            dimension_semantics=("arbitrary", "arbitrary"),
            vmem_limit_bytes=120 * 1024 * 1024,
        ),
    )(be, nb, xs, w3, gate_w, up_w, down_w)


NW = 32
TPW = S // NW
CH = 32

_SC_MESH = plsc.VectorSubcoreMesh(core_axis_name="c", subcore_axis_name="s")


def _sc_wid():
    return lax.axis_index("s") * 2 + lax.axis_index("c")


WV = 128


@functools.partial(
    pl.kernel,
    mesh=_SC_MESH,
    out_type=(
        jax.ShapeDtypeStruct((NSLOT, D), jnp.float32),
        jax.ShapeDtypeStruct((NSLOT, WV), jnp.float32),
    ),
    scratch_types=[
        pltpu.VMEM((TPW,), jnp.int32),
        pltpu.VMEM((TPW,), jnp.int32),
        pltpu.VMEM((TPW, D), jnp.float32),
        pltpu.VMEM((TPW, WV), jnp.float32),
        pltpu.VMEM((TPW, WV), jnp.float32),
        pltpu.SemaphoreType.DMA,
    ],
)
def _sc_scatter(x_hbm, s1_hbm, s2_hbm, w1_hbm, w2_hbm,
                out_hbm, wout_hbm, i1_v, i2_v, rows_v, w1_v, w2_v, sem):
    base = _sc_wid() * TPW
    pltpu.sync_copy(s1_hbm.at[pl.ds(base, TPW)], i1_v)
    pltpu.sync_copy(s2_hbm.at[pl.ds(base, TPW)], i2_v)
    pltpu.sync_copy(x_hbm.at[pl.ds(base, TPW)], rows_v)
    pltpu.sync_copy(w1_hbm.at[pl.ds(base, TPW)], w1_v)
    pltpu.sync_copy(w2_hbm.at[pl.ds(base, TPW)], w2_v)
    pltpu.async_copy(rows_v, out_hbm.at[i1_v], sem).wait()
    pltpu.async_copy(rows_v, out_hbm.at[i2_v], sem).wait()
    pltpu.async_copy(w1_v, wout_hbm.at[i1_v], sem).wait()
    pltpu.async_copy(w2_v, wout_hbm.at[i2_v], sem).wait()


@functools.partial(
    pl.kernel,
    mesh=_SC_MESH,
    out_type=jax.ShapeDtypeStruct((S, D), jnp.float32),
    scratch_types=[
        pltpu.VMEM((CH,), jnp.int32),
        pltpu.VMEM((CH,), jnp.int32),
        pltpu.VMEM((CH, D), jnp.float32),
        pltpu.VMEM((CH, D), jnp.float32),
        pltpu.SemaphoreType.DMA,
    ],
)
def _sc_combine(y_hbm, s1_hbm, s2_hbm, out_hbm, i1_v, i2_v, b1, b2, sem):
    base = _sc_wid() * TPW
    for c in range(TPW // CH):
        cb = base + c * CH
        pltpu.sync_copy(s1_hbm.at[pl.ds(cb, CH)], i1_v)
        pltpu.sync_copy(s2_hbm.at[pl.ds(cb, CH)], i2_v)
        pltpu.async_copy(y_hbm.at[i1_v], b1, sem).wait()
        pltpu.async_copy(y_hbm.at[i2_v], b2, sem).wait()

        def body(j, carry):
            for k in range(D // 16):
                sl = pl.ds(k * 16, 16)
                b1[j, sl] = b1[j, sl] + b2[j, sl]
            return carry

        lax.fori_loop(0, CH, body, 0)
        pltpu.sync_copy(b1, out_hbm.at[pl.ds(cb, CH)])


def kernel(hidden_states, router_w, gate_w, up_w, down_w):
    x = hidden_states.reshape(S, D)
    rw_pad = jnp.pad(router_w, ((0, 0), (0, LANES - E)))
    logits_pad, misc, be_pack = _routing(x, rw_pad)
    slot1 = misc[:, 0].astype(jnp.int32)
    slot2 = misc[:, 1].astype(jnp.int32)
    w1 = misc[:, 2]
    w2 = misc[:, 3]
    be = be_pack[:, 0].astype(jnp.int32)
    nb = be_pack[:1, 1].astype(jnp.int32)

    w1r = jnp.broadcast_to(w1[:, None], (S, WV))
    w2r = jnp.broadcast_to(w2[:, None], (S, WV))
    xs, ws = _sc_scatter(x, slot1, slot2, w1r, w2r)
    ys = _moe(be, nb, xs, ws, gate_w, up_w, down_w)
    out = _sc_combine(ys, slot1, slot2)

    return (out.reshape(1, S, D), logits_pad[:, :E].reshape(1, S, E))


# speedup vs baseline: 1.6269x; 1.0026x over previous
import functools
import jax
import jax.numpy as jnp
from jax import lax
from jax.experimental import pallas as pl
from jax.experimental.pallas import tpu as pltpu
from jax.experimental.pallas import tpu_sc as plsc

S, D, F, E = 2048, 1024, 4096, 8
A = S * 2
BLK = 256
NB = A // BLK + E - 1
NBP = 32
NSLOT = NB * BLK
FB = 512
NF = F // FB
LANES = 128
NEG = -1e30


def _routing_kernel(x_ref, rw_ref, logits_ref, misc_ref, be_ref,
                    oh1_s, oh2_s, r1_s, r2_s):
    x = x_ref[...]
    logits = jnp.dot(x, rw_ref[...], preferred_element_type=jnp.float32)
    logits_ref[...] = logits
    col = lax.broadcasted_iota(jnp.int32, (S, LANES), 1)
    neg = jnp.where(col < E, logits, NEG)
    m1 = jnp.max(neg, axis=1, keepdims=True)
    i1 = jnp.min(jnp.where(neg == m1, col, LANES), axis=1, keepdims=True)
    masked = jnp.where(col == i1, NEG, neg)
    m2 = jnp.max(masked, axis=1, keepdims=True)
    i2 = jnp.min(jnp.where(masked == m2, col, LANES), axis=1, keepdims=True)
    w1 = 1.0 / (1.0 + jnp.exp(m2 - m1))
    w2 = 1.0 - w1
    oh1 = (col == i1).astype(jnp.float32)
    oh2 = (col == i2).astype(jnp.float32)
    oh1_s[...] = oh1
    oh2_s[...] = oh2
    row128 = lax.broadcasted_iota(jnp.int32, (128, 128), 0)
    col128 = lax.broadcasted_iota(jnp.int32, (128, 128), 1)
    tlow = (row128 > col128).astype(jnp.float32)

    def mk_body(oh_s, r_s):
        def body(c, carry):
            ch = oh_s[pl.ds(c * 128, 128), :]
            rk = carry + jnp.dot(tlow, ch, preferred_element_type=jnp.float32)
            r_s[pl.ds(c * 128, 128), :] = jnp.sum(rk * ch, axis=1, keepdims=True)
            return carry + jnp.sum(ch, axis=0, keepdims=True)
        return body

    carry = lax.fori_loop(0, S // 128, mk_body(oh1_s, r1_s),
                          jnp.zeros((1, LANES), jnp.float32))
    counts = lax.fori_loop(0, S // 128, mk_body(oh2_s, r2_s), carry)
    blocks = jnp.floor((counts + (BLK - 1.0)) / BLK)
    ustrict = (row128 < col128).astype(jnp.float32)
    excl = jnp.dot(blocks, ustrict, preferred_element_type=jnp.float32)
    incl = excl + blocks
    pad_off = excl * BLK
    used = jnp.sum(blocks)
    slot1 = r1_s[...] + jnp.sum(oh1 * pad_off, axis=1, keepdims=True)
    slot2 = r2_s[...] + jnp.sum(oh2 * pad_off, axis=1, keepdims=True)
    misc = (jnp.where(col == 0, slot1, 0.0) + jnp.where(col == 1, slot2, 0.0)
            + jnp.where(col == 2, w1, 0.0) + jnp.where(col == 3, w2, 0.0))
    misc_ref[...] = misc
    jb = lax.broadcasted_iota(jnp.int32, (NBP, LANES), 0).astype(jnp.float32)
    jj = jnp.minimum(jb, used - 1.0)
    be_row = jnp.sum((jj >= incl).astype(jnp.float32), axis=1, keepdims=True)
    colb = lax.broadcasted_iota(jnp.int32, (NBP, LANES), 1)
    be_ref[...] = (jnp.where(colb == 0, be_row, 0.0)
                   + jnp.where(colb == 1, used, 0.0))


def _routing(x, rw_pad):
    return pl.pallas_call(
        _routing_kernel,
        out_shape=(
            jax.ShapeDtypeStruct((S, LANES), jnp.float32),
            jax.ShapeDtypeStruct((S, LANES), jnp.float32),
            jax.ShapeDtypeStruct((NBP, LANES), jnp.float32),
        ),
        scratch_shapes=[
            pltpu.VMEM((S, LANES), jnp.float32),
            pltpu.VMEM((S, LANES), jnp.float32),
            pltpu.VMEM((S, 1), jnp.float32),
            pltpu.VMEM((S, 1), jnp.float32),
        ],
    )(x, rw_pad)


def _moe_kernel(be_sref, nb_sref, x_ref, w_ref, gate_ref, up_ref, down_ref,
                out_ref):
    f = pl.program_id(0)
    i = pl.program_id(1)

    @pl.when(i < nb_sref[0])
    def _():
        rows = pl.ds(i * BLK, BLK)
        x = x_ref[rows, :].astype(jnp.bfloat16)
        g = jnp.dot(x, gate_ref[0].astype(jnp.bfloat16),
                    preferred_element_type=jnp.float32)
        u = jnp.dot(x, up_ref[0].astype(jnp.bfloat16),
                    preferred_element_type=jnp.float32)
        h = jax.nn.gelu(g, approximate=True) * u * w_ref[:, 0:1]
        contrib = jnp.dot(h.astype(jnp.bfloat16),
                          down_ref[0].astype(jnp.bfloat16),
                          preferred_element_type=jnp.float32)

        @pl.when(f == 0)
        def _():
            out_ref[rows, :] = contrib

        @pl.when(f > 0)
        def _():
            out_ref[rows, :] = out_ref[rows, :] + contrib


def _moe(be, nb, xs, w3, gate_w, up_w, down_w):
    grid_spec = pltpu.PrefetchScalarGridSpec(
        num_scalar_prefetch=2,
        grid=(NF, NB),
        in_specs=[
            pl.BlockSpec((NSLOT, D), lambda f, i, be, nb: (0, 0)),
            pl.BlockSpec((BLK, WV), lambda f, i, be, nb: (i, 0)),
            pl.BlockSpec((1, D, FB), lambda f, i, be, nb: (be[i], 0, f)),
            pl.BlockSpec((1, D, FB), lambda f, i, be, nb: (be[i], 0, f)),
            pl.BlockSpec((1, FB, D), lambda f, i, be, nb: (be[i], f, 0)),
        ],
        out_specs=pl.BlockSpec((NSLOT, D), lambda f, i, be, nb: (0, 0)),
    )
    return pl.pallas_call(
        _moe_kernel,
        grid_spec=grid_spec,
        out_shape=jax.ShapeDtypeStruct((NSLOT, D), jnp.float32),
        compiler_params=pltpu.CompilerParams(
            dimension_semantics=("arbitrary", "arbitrary"),
            vmem_limit_bytes=120 * 1024 * 1024,
        ),
    )(be, nb, xs, w3, gate_w, up_w, down_w)


NW = 32
TPW = S // NW
CH = 32

_SC_MESH = plsc.VectorSubcoreMesh(core_axis_name="c", subcore_axis_name="s")


def _sc_wid():
    return lax.axis_index("s") * 2 + lax.axis_index("c")


WV = 128


@functools.partial(
    pl.kernel,
    mesh=_SC_MESH,
    out_type=(
        jax.ShapeDtypeStruct((NSLOT, D), jnp.float32),
        jax.ShapeDtypeStruct((NSLOT, WV), jnp.float32),
    ),
    scratch_types=[
        pltpu.VMEM((TPW,), jnp.int32),
        pltpu.VMEM((TPW,), jnp.int32),
        pltpu.VMEM((TPW, D), jnp.float32),
        pltpu.VMEM((TPW, WV), jnp.float32),
        pltpu.VMEM((TPW, WV), jnp.float32),
        pltpu.SemaphoreType.DMA,
    ],
)
def _sc_scatter(x_hbm, s1_hbm, s2_hbm, w1_hbm, w2_hbm,
                out_hbm, wout_hbm, i1_v, i2_v, rows_v, w1_v, w2_v, sem):
    base = _sc_wid() * TPW
    pltpu.sync_copy(s1_hbm.at[pl.ds(base, TPW)], i1_v)
    pltpu.sync_copy(s2_hbm.at[pl.ds(base, TPW)], i2_v)
    pltpu.sync_copy(x_hbm.at[pl.ds(base, TPW)], rows_v)
    pltpu.sync_copy(w1_hbm.at[pl.ds(base, TPW)], w1_v)
    pltpu.sync_copy(w2_hbm.at[pl.ds(base, TPW)], w2_v)
    pltpu.async_copy(rows_v, out_hbm.at[i1_v], sem).wait()
    pltpu.async_copy(rows_v, out_hbm.at[i2_v], sem).wait()
    pltpu.async_copy(w1_v, wout_hbm.at[i1_v], sem).wait()
    pltpu.async_copy(w2_v, wout_hbm.at[i2_v], sem).wait()


@functools.partial(
    pl.kernel,
    mesh=_SC_MESH,
    out_type=jax.ShapeDtypeStruct((S, D), jnp.float32),
    scratch_types=[
        pltpu.VMEM((CH,), jnp.int32),
        pltpu.VMEM((CH,), jnp.int32),
        pltpu.VMEM((CH, D), jnp.float32),
        pltpu.VMEM((CH, D), jnp.float32),
        pltpu.SemaphoreType.DMA,
    ],
)
def _sc_combine(y_hbm, s1_hbm, s2_hbm, out_hbm, i1_v, i2_v, b1, b2, sem):
    base = _sc_wid() * TPW
    for c in range(TPW // CH):
        cb = base + c * CH
        pltpu.sync_copy(s1_hbm.at[pl.ds(cb, CH)], i1_v)
        pltpu.sync_copy(s2_hbm.at[pl.ds(cb, CH)], i2_v)
        pltpu.async_copy(y_hbm.at[i1_v], b1, sem).wait()
        pltpu.async_copy(y_hbm.at[i2_v], b2, sem).wait()

        def body(j, carry):
            for k in range(D // 16):
                sl = pl.ds(k * 16, 16)
                b1[j, sl] = b1[j, sl] + b2[j, sl]
            return carry

        lax.fori_loop(0, CH, body, 0)
        pltpu.sync_copy(b1, out_hbm.at[pl.ds(cb, CH)])


def kernel(hidden_states, router_w, gate_w, up_w, down_w):
    x = hidden_states.reshape(S, D)
    rw_pad = jnp.pad(router_w, ((0, 0), (0, LANES - E)))
    logits_pad, misc, be_pack = _routing(x, rw_pad)
    slot1 = misc[:, 0].astype(jnp.int32)
    slot2 = misc[:, 1].astype(jnp.int32)
    w1 = misc[:, 2]
    w2 = misc[:, 3]
    be = be_pack[:, 0].astype(jnp.int32)
    nb = be_pack[:1, 1].astype(jnp.int32)

    w1r = jnp.broadcast_to(w1[:, None], (S, WV))
    w2r = jnp.broadcast_to(w2[:, None], (S, WV))
    xs, ws = _sc_scatter(x, slot1, slot2, w1r, w2r)
    ys = _moe(be, nb, xs, ws, gate_w, up_w, down_w)
    out = _sc_combine(ys, slot1, slot2)

    return (out.reshape(1, S, D), logits_pad[:, :E].reshape(1, S, E))


# speedup vs baseline: 1.6481x; 1.0130x over previous
import functools
import jax
import jax.numpy as jnp
from jax import lax
from jax.experimental import pallas as pl
from jax.experimental.pallas import tpu as pltpu
from jax.experimental.pallas import tpu_sc as plsc

S, D, F, E = 2048, 1024, 4096, 8
A = S * 2
BLK = 256
NB = A // BLK + E - 1
NBP = 32
NSLOT = NB * BLK
FB = 256
NF = F // FB
LANES = 128
NEG = -1e30


def _routing_kernel(x_ref, rw_ref, logits_ref, misc_ref, be_ref,
                    oh1_s, oh2_s, r1_s, r2_s):
    x = x_ref[...]
    logits = jnp.dot(x, rw_ref[...], preferred_element_type=jnp.float32)
    logits_ref[...] = logits
    col = lax.broadcasted_iota(jnp.int32, (S, LANES), 1)
    neg = jnp.where(col < E, logits, NEG)
    m1 = jnp.max(neg, axis=1, keepdims=True)
    i1 = jnp.min(jnp.where(neg == m1, col, LANES), axis=1, keepdims=True)
    masked = jnp.where(col == i1, NEG, neg)
    m2 = jnp.max(masked, axis=1, keepdims=True)
    i2 = jnp.min(jnp.where(masked == m2, col, LANES), axis=1, keepdims=True)
    w1 = 1.0 / (1.0 + jnp.exp(m2 - m1))
    w2 = 1.0 - w1
    oh1 = (col == i1).astype(jnp.float32)
    oh2 = (col == i2).astype(jnp.float32)
    oh1_s[...] = oh1
    oh2_s[...] = oh2
    row128 = lax.broadcasted_iota(jnp.int32, (128, 128), 0)
    col128 = lax.broadcasted_iota(jnp.int32, (128, 128), 1)
    tlow = (row128 > col128).astype(jnp.float32)

    def mk_body(oh_s, r_s):
        def body(c, carry):
            ch = oh_s[pl.ds(c * 128, 128), :]
            rk = carry + jnp.dot(tlow, ch, preferred_element_type=jnp.float32)
            r_s[pl.ds(c * 128, 128), :] = jnp.sum(rk * ch, axis=1, keepdims=True)
            return carry + jnp.sum(ch, axis=0, keepdims=True)
        return body

    carry = lax.fori_loop(0, S // 128, mk_body(oh1_s, r1_s),
                          jnp.zeros((1, LANES), jnp.float32))
    counts = lax.fori_loop(0, S // 128, mk_body(oh2_s, r2_s), carry)
    blocks = jnp.floor((counts + (BLK - 1.0)) / BLK)
    ustrict = (row128 < col128).astype(jnp.float32)
    excl = jnp.dot(blocks, ustrict, preferred_element_type=jnp.float32)
    incl = excl + blocks
    pad_off = excl * BLK
    used = jnp.sum(blocks)
    slot1 = r1_s[...] + jnp.sum(oh1 * pad_off, axis=1, keepdims=True)
    slot2 = r2_s[...] + jnp.sum(oh2 * pad_off, axis=1, keepdims=True)
    misc = (jnp.where(col == 0, slot1, 0.0) + jnp.where(col == 1, slot2, 0.0)
            + jnp.where(col == 2, w1, 0.0) + jnp.where(col == 3, w2, 0.0))
    misc_ref[...] = misc
    has = (counts > 0.0).astype(jnp.float32)
    idxu = jnp.dot(has, ustrict, preferred_element_type=jnp.float32)
    nuniq = jnp.sum(has)
    jb = lax.broadcasted_iota(jnp.int32, (NBP, LANES), 0).astype(jnp.float32)
    jj = jnp.minimum(jb, nuniq - 1.0)
    sel_c = has * (idxu == jj).astype(jnp.float32)
    sel_r = has * (idxu == jb).astype(jnp.float32)
    colf = lax.broadcasted_iota(jnp.int32, (NBP, LANES), 1).astype(jnp.float32)
    e_j = jnp.sum(sel_c * colf, axis=1, keepdims=True)
    n_j = jnp.sum(sel_r * blocks, axis=1, keepdims=True)
    b_j = jnp.sum(sel_c * excl, axis=1, keepdims=True)
    colb = lax.broadcasted_iota(jnp.int32, (NBP, LANES), 1)
    be_ref[...] = (jnp.where(colb == 0, e_j, 0.0)
                   + jnp.where(colb == 1, n_j, 0.0)
                   + jnp.where(colb == 2, b_j, 0.0))


def _routing(x, rw_pad):
    return pl.pallas_call(
        _routing_kernel,
        out_shape=(
            jax.ShapeDtypeStruct((S, LANES), jnp.float32),
            jax.ShapeDtypeStruct((S, LANES), jnp.float32),
            jax.ShapeDtypeStruct((NBP, LANES), jnp.float32),
        ),
        scratch_shapes=[
            pltpu.VMEM((S, LANES), jnp.float32),
            pltpu.VMEM((S, LANES), jnp.float32),
            pltpu.VMEM((S, 1), jnp.float32),
            pltpu.VMEM((S, 1), jnp.float32),
        ],
    )(x, rw_pad)


def _moe_kernel(e_sref, n_sref, b_sref, x_ref, w_ref, gate_ref, up_ref,
                down_ref, out_ref):
    f = pl.program_id(0)
    j = pl.program_id(1)
    gate_b = gate_ref[0].astype(jnp.bfloat16)
    up_b = up_ref[0].astype(jnp.bfloat16)
    down_b = down_ref[0].astype(jnp.bfloat16)

    def body(t, carry):
        rows = pl.ds((b_sref[j] + t) * BLK, BLK)
        x = x_ref[rows, :].astype(jnp.bfloat16)
        g = jnp.dot(x, gate_b, preferred_element_type=jnp.float32)
        u = jnp.dot(x, up_b, preferred_element_type=jnp.float32)
        h = jax.nn.gelu(g, approximate=True) * u * w_ref[rows, 0:1]
        contrib = jnp.dot(h.astype(jnp.bfloat16), down_b,
                          preferred_element_type=jnp.float32)

        @pl.when(f == 0)
        def _():
            out_ref[rows, :] = contrib

        @pl.when(f > 0)
        def _():
            out_ref[rows, :] = out_ref[rows, :] + contrib

        return carry

    lax.fori_loop(0, n_sref[j], body, 0)


def _moe(e_arr, n_arr, b_arr, xs, ws, gate_w, up_w, down_w):
    grid_spec = pltpu.PrefetchScalarGridSpec(
        num_scalar_prefetch=3,
        grid=(NF, E),
        in_specs=[
            pl.BlockSpec((NSLOT, D), lambda f, j, e, n, b: (0, 0)),
            pl.BlockSpec((NSLOT, WV), lambda f, j, e, n, b: (0, 0)),
            pl.BlockSpec((1, D, FB), lambda f, j, e, n, b: (e[j], 0, f)),
            pl.BlockSpec((1, D, FB), lambda f, j, e, n, b: (e[j], 0, f)),
            pl.BlockSpec((1, FB, D), lambda f, j, e, n, b: (e[j], f, 0)),
        ],
        out_specs=pl.BlockSpec((NSLOT, D), lambda f, j, e, n, b: (0, 0)),
    )
    return pl.pallas_call(
        _moe_kernel,
        grid_spec=grid_spec,
        out_shape=jax.ShapeDtypeStruct((NSLOT, D), jnp.float32),
        compiler_params=pltpu.CompilerParams(
            dimension_semantics=("arbitrary", "arbitrary"),
            vmem_limit_bytes=63 * 1024 * 1024,
        ),
    )(e_arr, n_arr, b_arr, xs, ws, gate_w, up_w, down_w)


NW = 32
TPW = S // NW
CH = 32

_SC_MESH = plsc.VectorSubcoreMesh(core_axis_name="c", subcore_axis_name="s")


def _sc_wid():
    return lax.axis_index("s") * 2 + lax.axis_index("c")


WV = 128


@functools.partial(
    pl.kernel,
    mesh=_SC_MESH,
    out_type=(
        jax.ShapeDtypeStruct((NSLOT, D), jnp.float32),
        jax.ShapeDtypeStruct((NSLOT, WV), jnp.float32),
    ),
    scratch_types=[
        pltpu.VMEM((TPW,), jnp.int32),
        pltpu.VMEM((TPW,), jnp.int32),
        pltpu.VMEM((TPW, D), jnp.float32),
        pltpu.VMEM((TPW, WV), jnp.float32),
        pltpu.VMEM((TPW, WV), jnp.float32),
        pltpu.SemaphoreType.DMA,
    ],
)
def _sc_scatter(x_hbm, s1_hbm, s2_hbm, w1_hbm, w2_hbm,
                out_hbm, wout_hbm, i1_v, i2_v, rows_v, w1_v, w2_v, sem):
    base = _sc_wid() * TPW
    pltpu.sync_copy(s1_hbm.at[pl.ds(base, TPW)], i1_v)
    pltpu.sync_copy(s2_hbm.at[pl.ds(base, TPW)], i2_v)
    pltpu.sync_copy(x_hbm.at[pl.ds(base, TPW)], rows_v)
    pltpu.sync_copy(w1_hbm.at[pl.ds(base, TPW)], w1_v)
    pltpu.sync_copy(w2_hbm.at[pl.ds(base, TPW)], w2_v)
    pltpu.async_copy(rows_v, out_hbm.at[i1_v], sem).wait()
    pltpu.async_copy(rows_v, out_hbm.at[i2_v], sem).wait()
    pltpu.async_copy(w1_v, wout_hbm.at[i1_v], sem).wait()
    pltpu.async_copy(w2_v, wout_hbm.at[i2_v], sem).wait()


@functools.partial(
    pl.kernel,
    mesh=_SC_MESH,
    out_type=jax.ShapeDtypeStruct((S, D), jnp.float32),
    scratch_types=[
        pltpu.VMEM((CH,), jnp.int32),
        pltpu.VMEM((CH,), jnp.int32),
        pltpu.VMEM((CH, D), jnp.float32),
        pltpu.VMEM((CH, D), jnp.float32),
        pltpu.SemaphoreType.DMA,
    ],
)
def _sc_combine(y_hbm, s1_hbm, s2_hbm, out_hbm, i1_v, i2_v, b1, b2, sem):
    base = _sc_wid() * TPW
    for c in range(TPW // CH):
        cb = base + c * CH
        pltpu.sync_copy(s1_hbm.at[pl.ds(cb, CH)], i1_v)
        pltpu.sync_copy(s2_hbm.at[pl.ds(cb, CH)], i2_v)
        pltpu.async_copy(y_hbm.at[i1_v], b1, sem).wait()
        pltpu.async_copy(y_hbm.at[i2_v], b2, sem).wait()

        def body(j, carry):
            for k in range(D // 16):
                sl = pl.ds(k * 16, 16)
                b1[j, sl] = b1[j, sl] + b2[j, sl]
            return carry

        lax.fori_loop(0, CH, body, 0)
        pltpu.sync_copy(b1, out_hbm.at[pl.ds(cb, CH)])


def kernel(hidden_states, router_w, gate_w, up_w, down_w):
    x = hidden_states.reshape(S, D)
    rw_pad = jnp.pad(router_w, ((0, 0), (0, LANES - E)))
    logits_pad, misc, be_pack = _routing(x, rw_pad)
    slot1 = misc[:, 0].astype(jnp.int32)
    slot2 = misc[:, 1].astype(jnp.int32)
    w1 = misc[:, 2]
    w2 = misc[:, 3]
    e_arr = be_pack[:E, 0].astype(jnp.int32)
    n_arr = be_pack[:E, 1].astype(jnp.int32)
    b_arr = be_pack[:E, 2].astype(jnp.int32)

    w1r = jnp.broadcast_to(w1[:, None], (S, WV))
    w2r = jnp.broadcast_to(w2[:, None], (S, WV))
    xs, ws = _sc_scatter(x, slot1, slot2, w1r, w2r)
    ys = _moe(e_arr, n_arr, b_arr, xs, ws, gate_w, up_w, down_w)
    out = _sc_combine(ys, slot1, slot2)

    return (out.reshape(1, S, D), logits_pad[:, :E].reshape(1, S, E))
